# baseline (speedup 1.0000x reference)
import functools

import jax
import jax.numpy as jnp
from jax import lax
from jax.experimental import pallas as pl
from jax.experimental.pallas import tpu as pltpu
from jax.experimental.pallas import tpu_sc as plsc

N, H, W = 16, 224, 224
C = 96
P = H * W
L = 16
NC, NS = 2, 16
NW = NC * NS

NSC = 4
WPI = NW // NSC
SPW = P // WPI
TP = 128
TASKS = SPW // TP
CHUNKS = TP // L

HBT = 112


def _sc_onehot(xs):
    mesh = plsc.VectorSubcoreMesh(core_axis_name="c", subcore_axis_name="s")

    @functools.partial(
        pl.kernel,
        mesh=mesh,
        compiler_params=pltpu.CompilerParams(
            use_tc_tiling_on_sc=False, needs_layout_passes=False
        ),
        out_type=jax.ShapeDtypeStruct((NSC, C, P), jnp.float32),
        scratch_types=[
            pltpu.VMEM((SPW,), jnp.int32),
            pltpu.VMEM((C, TP), jnp.float32),
            pltpu.VMEM((C, TP), jnp.float32),
            pltpu.SemaphoreType.DMA,
            pltpu.SemaphoreType.DMA,
        ],
    )
    def k(x_hbm, out_hbm, x_v, oh0, oh1, sem0, sem1):
        wid = lax.axis_index("s") * NC + lax.axis_index("c")
        n = wid // WPI
        p_base = (wid % WPI) * SPW

        zeros = jnp.zeros((L,), jnp.float32)
        ones = jnp.ones((L,), jnp.float32)
        lane = lax.broadcasted_iota(jnp.int32, (L,), 0)
        bufs = (oh0, oh1)
        sems = (sem0, sem1)

        pltpu.sync_copy(x_hbm.at[pl.ds(wid * SPW, SPW)], x_v)

        def zbody(c, carry):
            for buf in bufs:
                for j in range(CHUNKS):
                    buf[c, pl.ds(j * L, L)] = zeros
            return carry

        lax.fori_loop(0, C, zbody, 0)

        def scatter_task(buf, i, val_vec):
            for j in range(CHUNKS):
                vals = x_v[pl.ds(i * TP + j * L, L)]
                plsc.store_scatter(buf, [vals, lane + j * L], val_vec)

        def do_task(buf, sem, i, first):
            @pl.when(jnp.logical_not(first))
            def _():
                pltpu.make_async_copy(
                    buf, out_hbm.at[n, :, pl.ds(p_base, TP)], sem
                ).wait()
                scatter_task(buf, i - 2, zeros)

            scatter_task(buf, i, ones)
            pltpu.async_copy(
                buf, out_hbm.at[n, :, pl.ds(p_base + i * TP, TP)], sem
            )

        def gbody(g, carry):
            for b in range(2):
                do_task(bufs[b], sems[b], g * 2 + b, g == 0)
            return carry

        lax.fori_loop(0, TASKS // 2, gbody, 0)
        do_task(bufs[0], sems[0], TASKS - 1, jnp.bool_(False))

        pltpu.make_async_copy(
            bufs[0], out_hbm.at[n, :, pl.ds(p_base, TP)], sems[0]
        ).wait()
        pltpu.make_async_copy(
            bufs[1], out_hbm.at[n, :, pl.ds(p_base, TP)], sems[1]
        ).wait()

    return k(xs)


def _tc_body(x_ref, o_ref):
    x = x_ref[0]
    cio = jax.lax.broadcasted_iota(jnp.int32, (C, HBT, W), 0)
    o_ref[0] = jnp.where(cio == x[None], 1.0, 0.0).astype(jnp.float32)


def _tc_onehot(x):
    n = x.shape[0]
    grid = (n, H // HBT)
    return pl.pallas_call(
        _tc_body,
        grid=grid,
        in_specs=[pl.BlockSpec((1, HBT, W), lambda i, j: (i, j, 0))],
        out_specs=pl.BlockSpec((1, C, HBT, W), lambda i, j: (i, 0, j, 0)),
        out_shape=jax.ShapeDtypeStruct((n, C, H, W), jnp.float32),
    )(x)


def kernel(x):
    sc_part = _sc_onehot(x[:NSC].reshape(NSC * P)).reshape(NSC, C, H, W)
    tc_part = _tc_onehot(x[NSC:])
    return jnp.concatenate([sc_part, tc_part], axis=0)
